# Initial kernel scaffold; baseline (speedup 1.0000x reference)
import jax
import jax.numpy as jnp
import numpy as np
from jax.experimental import pallas as pl
from jax.experimental.pallas import tpu as pltpu

H = W = 512
VIEW_TAN = float(np.tan(30.0 * np.pi / 180.0))
NEAR = 0.1
BIG = 1e10
F = 4096
TH = 64
NT = H // TH


def _raster_kernel(coef_ref, refimg_ref, out_ref, zb, rb, gb, bb):
    i = pl.program_id(0)
    colf = jax.lax.broadcasted_iota(jnp.float32, (TH, W), 1)
    rowf = jax.lax.broadcasted_iota(jnp.float32, (TH, W), 0)
    xv = (colf + 0.5) * (2.0 / W) - 1.0
    ytop = 1.0 - (i.astype(jnp.float32) * TH + 0.5) * (2.0 / H)
    yv = ytop - rowf * (2.0 / H)
    ylo = ytop - (TH - 1) * (2.0 / H)
    yhi = ytop

    zb[...] = jnp.full((TH, W), BIG, dtype=jnp.float32)
    rb[...] = jnp.zeros((TH, W), dtype=jnp.float32)
    gb[...] = jnp.zeros((TH, W), dtype=jnp.float32)
    bb[...] = jnp.zeros((TH, W), dtype=jnp.float32)

    def body(f, carry):
        @pl.when((coef_ref[15, f] <= yhi) & (coef_ref[16, f] >= ylo))
        def _():
            w0 = xv * coef_ref[0, f] + (yv * coef_ref[1, f] + coef_ref[2, f])
            w1 = xv * coef_ref[3, f] + (yv * coef_ref[4, f] + coef_ref[5, f])
            w2 = xv * coef_ref[6, f] + (yv * coef_ref[7, f] + coef_ref[8, f])
            zv = xv * coef_ref[9, f] + (yv * coef_ref[10, f] + coef_ref[11, f])
            m = jnp.minimum(jnp.minimum(w0, w1), w2)
            ok = (m >= 0.0) & (zv > NEAR)
            zm = jnp.where(ok, zv, BIG)
            zold = zb[...]
            upd = zm < zold
            zb[...] = jnp.where(upd, zm, zold)
            rb[...] = jnp.where(upd, coef_ref[12, f], rb[...])
            gb[...] = jnp.where(upd, coef_ref[13, f], gb[...])
            bb[...] = jnp.where(upd, coef_ref[14, f], bb[...])
        return carry

    jax.lax.fori_loop(0, F, body, 0)

    dr = rb[...] - refimg_ref[0]
    dg = gb[...] - refimg_ref[1]
    db = bb[...] - refimg_ref[2]
    sse = dr * dr + dg * dg + db * db
    out_ref[...] = jnp.sum(sse, axis=0, keepdims=True)[None]


def _cross2(ax, ay, bx, by):
    return ax * by - ay * bx


def kernel(vertices, faces, textures, camera_position, camera_up, image_ref):
    eye = camera_position

    def _norm(x):
        return x / (jnp.linalg.norm(x) + 1e-12)

    z_axis = _norm(-eye)
    x_axis = _norm(jnp.cross(_norm(camera_up), z_axis))
    y_axis = jnp.cross(z_axis, x_axis)
    R = jnp.stack([x_axis, y_axis, z_axis], axis=0)
    v_cam = (vertices[0] - eye) @ R.T
    zc = v_cam[:, 2]
    proj = v_cam[:, :2] / (zc[:, None] * VIEW_TAN + 1e-12)

    cols = textures[0].mean(axis=(1, 2, 3))
    tri = proj[faces[0]]
    tz = zc[faces[0]]
    p0x, p0y = tri[:, 0, 0], tri[:, 0, 1]
    p1x, p1y = tri[:, 1, 0], tri[:, 1, 1]
    p2x, p2y = tri[:, 2, 0], tri[:, 2, 1]

    area = _cross2(p1x - p0x, p1y - p0y, p2x - p0x, p2y - p0y)
    valid = jnp.abs(area) > 1e-8
    safe = jnp.where(valid, area, 1.0)
    s = jnp.sign(area)

    A0 = -(p2y - p1y); B0 = p2x - p1x; C0 = (p2y - p1y) * p1x - (p2x - p1x) * p1y
    A1 = -(p0y - p2y); B1 = p0x - p2x; C1 = (p0y - p2y) * p2x - (p0x - p2x) * p2y
    A2 = -(p1y - p0y); B2 = p1x - p0x; C2 = (p1y - p0y) * p0x - (p1x - p0x) * p0y

    inv = 1.0 / safe
    Az = (A0 * tz[:, 0] + A1 * tz[:, 1] + A2 * tz[:, 2]) * inv
    Bz = (B0 * tz[:, 0] + B1 * tz[:, 1] + B2 * tz[:, 2]) * inv
    Cz = (C0 * tz[:, 0] + C1 * tz[:, 1] + C2 * tz[:, 2]) * inv
    Az = jnp.where(valid, Az, 0.0)
    Bz = jnp.where(valid, Bz, 0.0)
    Cz = jnp.where(valid, Cz, 0.0)

    fymin = jnp.minimum(jnp.minimum(p0y, p1y), p2y)
    fymax = jnp.maximum(jnp.maximum(p0y, p1y), p2y)

    coef = jnp.stack([
        s * A0, s * B0, s * C0,
        s * A1, s * B1, s * C1,
        s * A2, s * B2, s * C2,
        Az, Bz, Cz,
        cols[:, 0], cols[:, 1], cols[:, 2],
        fymin, fymax,
    ], axis=0)

    refimg = image_ref[0, ::-1]

    out = pl.pallas_call(
        _raster_kernel,
        grid=(NT,),
        in_specs=[
            pl.BlockSpec(memory_space=pltpu.SMEM),
            pl.BlockSpec((3, TH, W), lambda i: (0, i, 0)),
        ],
        out_specs=pl.BlockSpec((1, 1, W), lambda i: (i, 0, 0)),
        out_shape=jax.ShapeDtypeStruct((NT, 1, W), jnp.float32),
        scratch_shapes=[pltpu.VMEM((TH, W), jnp.float32)] * 4,
        compiler_params=pltpu.CompilerParams(
            dimension_semantics=("core_parallel",),
            vmem_limit_bytes=50 * 1024 * 1024,
        ),
    )(coef, refimg)
    return jnp.sum(out)


# speedup vs baseline: 65.9726x; 65.9726x over previous
import jax
import jax.numpy as jnp
import numpy as np
from jax.experimental import pallas as pl
from jax.experimental.pallas import tpu as pltpu

H = W = 512
VIEW_TAN = float(np.tan(30.0 * np.pi / 180.0))
NEAR = 0.1
BIG = 1e10
F = 4096
TH = 64
NT = H // TH


def _raster_kernel(coef_ref, refimg_ref, out_ref, zb, rb, gb, bb):
    i = pl.program_id(0)
    colf = jax.lax.broadcasted_iota(jnp.int32, (TH, W), 1).astype(jnp.float32)
    rowf = jax.lax.broadcasted_iota(jnp.int32, (TH, W), 0).astype(jnp.float32)
    xv = (colf + 0.5) * (2.0 / W) - 1.0
    ytop = 1.0 - (i.astype(jnp.float32) * TH + 0.5) * (2.0 / H)
    yv = ytop - rowf * (2.0 / H)
    ylo = ytop - (TH - 1) * (2.0 / H)
    yhi = ytop

    zb[...] = jnp.full((TH, W), BIG, dtype=jnp.float32)
    rb[...] = jnp.zeros((TH, W), dtype=jnp.float32)
    gb[...] = jnp.zeros((TH, W), dtype=jnp.float32)
    bb[...] = jnp.zeros((TH, W), dtype=jnp.float32)

    def body(f, carry):
        @pl.when((coef_ref[15, f] <= yhi) & (coef_ref[16, f] >= ylo))
        def _():
            w0 = xv * coef_ref[0, f] + (yv * coef_ref[1, f] + coef_ref[2, f])
            w1 = xv * coef_ref[3, f] + (yv * coef_ref[4, f] + coef_ref[5, f])
            w2 = xv * coef_ref[6, f] + (yv * coef_ref[7, f] + coef_ref[8, f])
            zv = xv * coef_ref[9, f] + (yv * coef_ref[10, f] + coef_ref[11, f])
            m = jnp.minimum(jnp.minimum(w0, w1), w2)
            ok = (m >= 0.0) & (zv > NEAR)
            zm = jnp.where(ok, zv, BIG)
            zold = zb[...]
            upd = zm < zold
            zb[...] = jnp.where(upd, zm, zold)
            rb[...] = jnp.where(upd, coef_ref[12, f], rb[...])
            gb[...] = jnp.where(upd, coef_ref[13, f], gb[...])
            bb[...] = jnp.where(upd, coef_ref[14, f], bb[...])
        return carry

    jax.lax.fori_loop(0, F, body, 0)

    dr = rb[...] - refimg_ref[0]
    dg = gb[...] - refimg_ref[1]
    db = bb[...] - refimg_ref[2]
    sse = dr * dr + dg * dg + db * db
    out_ref[...] = jnp.sum(sse, axis=0, keepdims=True)[None]


def _cross2(ax, ay, bx, by):
    return ax * by - ay * bx


def kernel(vertices, faces, textures, camera_position, camera_up, image_ref):
    eye = camera_position

    def _norm(x):
        return x / (jnp.linalg.norm(x) + 1e-12)

    z_axis = _norm(-eye)
    x_axis = _norm(jnp.cross(_norm(camera_up), z_axis))
    y_axis = jnp.cross(z_axis, x_axis)
    R = jnp.stack([x_axis, y_axis, z_axis], axis=0)
    v_cam = (vertices[0] - eye) @ R.T
    zc = v_cam[:, 2]
    proj = v_cam[:, :2] / (zc[:, None] * VIEW_TAN + 1e-12)

    cols = textures[0].mean(axis=(1, 2, 3))
    tri = proj[faces[0]]
    tz = zc[faces[0]]
    p0x, p0y = tri[:, 0, 0], tri[:, 0, 1]
    p1x, p1y = tri[:, 1, 0], tri[:, 1, 1]
    p2x, p2y = tri[:, 2, 0], tri[:, 2, 1]

    area = _cross2(p1x - p0x, p1y - p0y, p2x - p0x, p2y - p0y)
    valid = jnp.abs(area) > 1e-8
    safe = jnp.where(valid, area, 1.0)
    s = jnp.sign(area)

    A0 = -(p2y - p1y); B0 = p2x - p1x; C0 = (p2y - p1y) * p1x - (p2x - p1x) * p1y
    A1 = -(p0y - p2y); B1 = p0x - p2x; C1 = (p0y - p2y) * p2x - (p0x - p2x) * p2y
    A2 = -(p1y - p0y); B2 = p1x - p0x; C2 = (p1y - p0y) * p0x - (p1x - p0x) * p0y

    inv = 1.0 / safe
    Az = (A0 * tz[:, 0] + A1 * tz[:, 1] + A2 * tz[:, 2]) * inv
    Bz = (B0 * tz[:, 0] + B1 * tz[:, 1] + B2 * tz[:, 2]) * inv
    Cz = (C0 * tz[:, 0] + C1 * tz[:, 1] + C2 * tz[:, 2]) * inv
    Az = jnp.where(valid, Az, 0.0)
    Bz = jnp.where(valid, Bz, 0.0)
    Cz = jnp.where(valid, Cz, 0.0)

    fymin = jnp.minimum(jnp.minimum(p0y, p1y), p2y)
    fymax = jnp.maximum(jnp.maximum(p0y, p1y), p2y)

    coef = jnp.stack([
        s * A0, s * B0, s * C0,
        s * A1, s * B1, s * C1,
        s * A2, s * B2, s * C2,
        Az, Bz, Cz,
        cols[:, 0], cols[:, 1], cols[:, 2],
        fymin, fymax,
    ], axis=0)

    refimg = image_ref[0, ::-1]

    out = pl.pallas_call(
        _raster_kernel,
        grid=(NT,),
        in_specs=[
            pl.BlockSpec(memory_space=pltpu.SMEM),
            pl.BlockSpec((3, TH, W), lambda i: (0, i, 0)),
        ],
        out_specs=pl.BlockSpec((1, 1, W), lambda i: (i, 0, 0)),
        out_shape=jax.ShapeDtypeStruct((NT, 1, W), jnp.float32),
        scratch_shapes=[pltpu.VMEM((TH, W), jnp.float32)] * 4,
        compiler_params=pltpu.CompilerParams(
            dimension_semantics=("parallel",),
            vmem_limit_bytes=50 * 1024 * 1024,
        ),
    )(coef, refimg)
    return jnp.sum(out)


# speedup vs baseline: 71.8555x; 1.0892x over previous
import jax
import jax.numpy as jnp
import numpy as np
from jax.experimental import pallas as pl
from jax.experimental.pallas import tpu as pltpu

H = W = 512
VIEW_TAN = float(np.tan(30.0 * np.pi / 180.0))
NEAR = 0.1
BIG = 1e10
F = 4096
TH = 64
NT = H // TH


def _raster_kernel(coef_ref, refimg_ref, out_ref, zb, rb, gb, bb):
    i = pl.program_id(0)
    colf = jax.lax.broadcasted_iota(jnp.int32, (TH, W), 1).astype(jnp.float32)
    rowf = jax.lax.broadcasted_iota(jnp.int32, (TH, W), 0).astype(jnp.float32)
    xv = (colf + 0.5) * (2.0 / W) - 1.0
    ytop = 1.0 - (i.astype(jnp.float32) * TH + 0.5) * (2.0 / H)
    yv = ytop - rowf * (2.0 / H)
    ylo = ytop - (TH - 1) * (2.0 / H)
    yhi = ytop

    zb[...] = jnp.full((TH, W), BIG, dtype=jnp.float32)
    rb[...] = jnp.zeros((TH, W), dtype=jnp.float32)
    gb[...] = jnp.zeros((TH, W), dtype=jnp.float32)
    bb[...] = jnp.zeros((TH, W), dtype=jnp.float32)

    def face_work(f):
        @pl.when((coef_ref[15, f] <= yhi) & (coef_ref[16, f] >= ylo))
        def _():
            w0 = xv * coef_ref[0, f] + (yv * coef_ref[1, f] + coef_ref[2, f])
            w1 = xv * coef_ref[3, f] + (yv * coef_ref[4, f] + coef_ref[5, f])
            w2 = xv * coef_ref[6, f] + (yv * coef_ref[7, f] + coef_ref[8, f])
            zv = xv * coef_ref[9, f] + (yv * coef_ref[10, f] + coef_ref[11, f])
            m = jnp.minimum(jnp.minimum(w0, w1), w2)
            zold = zb[...]
            t = jnp.minimum(jnp.minimum(m, zv), zold - zv)
            upd = t > 0.0
            zb[...] = jnp.where(upd, zv, zold)
            rb[...] = jnp.where(upd, coef_ref[12, f], rb[...])
            gb[...] = jnp.where(upd, coef_ref[13, f], gb[...])
            bb[...] = jnp.where(upd, coef_ref[14, f], bb[...])

    def body(k, carry):
        face_work(2 * k)
        face_work(2 * k + 1)
        return carry

    jax.lax.fori_loop(0, F // 2, body, 0)

    dr = rb[...] - refimg_ref[0]
    dg = gb[...] - refimg_ref[1]
    db = bb[...] - refimg_ref[2]
    sse = dr * dr + dg * dg + db * db
    out_ref[...] = jnp.sum(sse, axis=0, keepdims=True)[None]


def _cross2(ax, ay, bx, by):
    return ax * by - ay * bx


def kernel(vertices, faces, textures, camera_position, camera_up, image_ref):
    eye = camera_position

    def _norm(x):
        return x / (jnp.linalg.norm(x) + 1e-12)

    z_axis = _norm(-eye)
    x_axis = _norm(jnp.cross(_norm(camera_up), z_axis))
    y_axis = jnp.cross(z_axis, x_axis)
    R = jnp.stack([x_axis, y_axis, z_axis], axis=0)
    v_cam = (vertices[0] - eye) @ R.T
    zc = v_cam[:, 2]
    proj = v_cam[:, :2] / (zc[:, None] * VIEW_TAN + 1e-12)

    cols = textures[0].mean(axis=(1, 2, 3))
    tri = proj[faces[0]]
    tz = zc[faces[0]]
    p0x, p0y = tri[:, 0, 0], tri[:, 0, 1]
    p1x, p1y = tri[:, 1, 0], tri[:, 1, 1]
    p2x, p2y = tri[:, 2, 0], tri[:, 2, 1]

    area = _cross2(p1x - p0x, p1y - p0y, p2x - p0x, p2y - p0y)
    valid = jnp.abs(area) > 1e-8
    safe = jnp.where(valid, area, 1.0)
    s = jnp.sign(area)

    A0 = -(p2y - p1y); B0 = p2x - p1x; C0 = (p2y - p1y) * p1x - (p2x - p1x) * p1y
    A1 = -(p0y - p2y); B1 = p0x - p2x; C1 = (p0y - p2y) * p2x - (p0x - p2x) * p2y
    A2 = -(p1y - p0y); B2 = p1x - p0x; C2 = (p1y - p0y) * p0x - (p1x - p0x) * p0y

    inv = 1.0 / safe
    Az = (A0 * tz[:, 0] + A1 * tz[:, 1] + A2 * tz[:, 2]) * inv
    Bz = (B0 * tz[:, 0] + B1 * tz[:, 1] + B2 * tz[:, 2]) * inv
    Cz = (C0 * tz[:, 0] + C1 * tz[:, 1] + C2 * tz[:, 2]) * inv
    Az = jnp.where(valid, Az, 0.0)
    Bz = jnp.where(valid, Bz, 0.0)
    Cz = jnp.where(valid, Cz - NEAR, -1.0)

    fymin = jnp.minimum(jnp.minimum(p0y, p1y), p2y)
    fymax = jnp.maximum(jnp.maximum(p0y, p1y), p2y)

    coef = jnp.stack([
        s * A0, s * B0, s * C0,
        s * A1, s * B1, s * C1,
        s * A2, s * B2, s * C2,
        Az, Bz, Cz,
        cols[:, 0], cols[:, 1], cols[:, 2],
        fymin, fymax,
    ], axis=0)

    refimg = image_ref[0, ::-1]

    out = pl.pallas_call(
        _raster_kernel,
        grid=(NT,),
        in_specs=[
            pl.BlockSpec(memory_space=pltpu.SMEM),
            pl.BlockSpec((3, TH, W), lambda i: (0, i, 0)),
        ],
        out_specs=pl.BlockSpec((1, 1, W), lambda i: (i, 0, 0)),
        out_shape=jax.ShapeDtypeStruct((NT, 1, W), jnp.float32),
        scratch_shapes=[pltpu.VMEM((TH, W), jnp.float32)] * 4,
        compiler_params=pltpu.CompilerParams(
            dimension_semantics=("parallel",),
            vmem_limit_bytes=50 * 1024 * 1024,
        ),
    )(coef, refimg)
    return jnp.sum(out)


# speedup vs baseline: 75.8539x; 1.0556x over previous
import jax
import jax.numpy as jnp
import numpy as np
from jax.experimental import pallas as pl
from jax.experimental.pallas import tpu as pltpu

H = W = 512
VIEW_TAN = float(np.tan(30.0 * np.pi / 180.0))
NEAR = 0.1
BIG = 1e10
F = 4096
TH = 64
NT = H // TH


def _raster_kernel(coef_ref, pairbb_ref, refimg_ref, out_ref, zb, rb, gb, bb):
    i = pl.program_id(0)
    colf = jax.lax.broadcasted_iota(jnp.int32, (TH, W), 1).astype(jnp.float32)
    rowf = jax.lax.broadcasted_iota(jnp.int32, (TH, W), 0).astype(jnp.float32)
    xv = (colf + 0.5) * (2.0 / W) - 1.0
    ytop = 1.0 - (i.astype(jnp.float32) * TH + 0.5) * (2.0 / H)
    yv = ytop - rowf * (2.0 / H)
    ylo = ytop - (TH - 1) * (2.0 / H)
    yhi = ytop

    zb[...] = jnp.full((TH, W), BIG, dtype=jnp.float32)
    rb[...] = jnp.zeros((TH, W), dtype=jnp.float32)
    gb[...] = jnp.zeros((TH, W), dtype=jnp.float32)
    bb[...] = jnp.zeros((TH, W), dtype=jnp.float32)

    def face_work(f):
        w0 = xv * coef_ref[0, f] + (yv * coef_ref[1, f] + coef_ref[2, f])
        w1 = xv * coef_ref[3, f] + (yv * coef_ref[4, f] + coef_ref[5, f])
        w2 = xv * coef_ref[6, f] + (yv * coef_ref[7, f] + coef_ref[8, f])
        zv = xv * coef_ref[9, f] + (yv * coef_ref[10, f] + coef_ref[11, f])
        m = jnp.minimum(jnp.minimum(w0, w1), w2)
        zold = zb[...]
        t = jnp.minimum(jnp.minimum(m, zv), zold - zv)
        upd = t > 0.0
        zb[...] = jnp.where(upd, zv, zold)
        rb[...] = jnp.where(upd, coef_ref[12, f], rb[...])
        gb[...] = jnp.where(upd, coef_ref[13, f], gb[...])
        bb[...] = jnp.where(upd, coef_ref[14, f], bb[...])

    def body(k, carry):
        @pl.when((pairbb_ref[0, k] <= yhi) & (pairbb_ref[1, k] >= ylo))
        def _():
            face_work(2 * k)
            face_work(2 * k + 1)
        return carry

    jax.lax.fori_loop(0, F // 2, body, 0)

    dr = rb[...] - refimg_ref[0]
    dg = gb[...] - refimg_ref[1]
    db = bb[...] - refimg_ref[2]
    sse = dr * dr + dg * dg + db * db
    out_ref[...] = jnp.sum(sse, axis=0, keepdims=True)[None]


def _cross2(ax, ay, bx, by):
    return ax * by - ay * bx


def kernel(vertices, faces, textures, camera_position, camera_up, image_ref):
    eye = camera_position

    def _norm(x):
        return x / (jnp.linalg.norm(x) + 1e-12)

    z_axis = _norm(-eye)
    x_axis = _norm(jnp.cross(_norm(camera_up), z_axis))
    y_axis = jnp.cross(z_axis, x_axis)
    R = jnp.stack([x_axis, y_axis, z_axis], axis=0)
    v_cam = (vertices[0] - eye) @ R.T
    zc = v_cam[:, 2]
    proj = v_cam[:, :2] / (zc[:, None] * VIEW_TAN + 1e-12)

    cols = textures[0].mean(axis=(1, 2, 3))
    tri = proj[faces[0]]
    tz = zc[faces[0]]
    p0x, p0y = tri[:, 0, 0], tri[:, 0, 1]
    p1x, p1y = tri[:, 1, 0], tri[:, 1, 1]
    p2x, p2y = tri[:, 2, 0], tri[:, 2, 1]

    area = _cross2(p1x - p0x, p1y - p0y, p2x - p0x, p2y - p0y)
    valid = jnp.abs(area) > 1e-8
    safe = jnp.where(valid, area, 1.0)
    s = jnp.sign(area)

    A0 = -(p2y - p1y); B0 = p2x - p1x; C0 = (p2y - p1y) * p1x - (p2x - p1x) * p1y
    A1 = -(p0y - p2y); B1 = p0x - p2x; C1 = (p0y - p2y) * p2x - (p0x - p2x) * p2y
    A2 = -(p1y - p0y); B2 = p1x - p0x; C2 = (p1y - p0y) * p0x - (p1x - p0x) * p0y

    inv = 1.0 / safe
    Az = (A0 * tz[:, 0] + A1 * tz[:, 1] + A2 * tz[:, 2]) * inv
    Bz = (B0 * tz[:, 0] + B1 * tz[:, 1] + B2 * tz[:, 2]) * inv
    Cz = (C0 * tz[:, 0] + C1 * tz[:, 1] + C2 * tz[:, 2]) * inv
    Az = jnp.where(valid, Az, 0.0)
    Bz = jnp.where(valid, Bz, 0.0)
    Cz = jnp.where(valid, Cz - NEAR, -1.0)

    fymin = jnp.minimum(jnp.minimum(p0y, p1y), p2y)
    fymax = jnp.maximum(jnp.maximum(p0y, p1y), p2y)

    coef = jnp.stack([
        s * A0, s * B0, s * C0,
        s * A1, s * B1, s * C1,
        s * A2, s * B2, s * C2,
        Az, Bz, Cz,
        cols[:, 0], cols[:, 1], cols[:, 2],
    ], axis=0)

    order = jnp.argsort(fymin + fymax)
    coef = coef[:, order]
    fymin = fymin[order]
    fymax = fymax[order]
    pairbb = jnp.stack([
        jnp.minimum(fymin[0::2], fymin[1::2]),
        jnp.maximum(fymax[0::2], fymax[1::2]),
    ], axis=0)

    refimg = image_ref[0, ::-1]

    out = pl.pallas_call(
        _raster_kernel,
        grid=(NT,),
        in_specs=[
            pl.BlockSpec(memory_space=pltpu.SMEM),
            pl.BlockSpec(memory_space=pltpu.SMEM),
            pl.BlockSpec((3, TH, W), lambda i: (0, i, 0)),
        ],
        out_specs=pl.BlockSpec((1, 1, W), lambda i: (i, 0, 0)),
        out_shape=jax.ShapeDtypeStruct((NT, 1, W), jnp.float32),
        scratch_shapes=[pltpu.VMEM((TH, W), jnp.float32)] * 4,
        compiler_params=pltpu.CompilerParams(
            dimension_semantics=("parallel",),
            vmem_limit_bytes=50 * 1024 * 1024,
        ),
    )(coef, pairbb, refimg)
    return jnp.sum(out)
